# Initial kernel scaffold; baseline (speedup 1.0000x reference)
import jax
import jax.numpy as jnp
from jax.experimental import pallas as pl


def kernel(x, edge_index, sampled_nodes, W1, W2):
    raise NotImplementedError("write your pallas kernel here")


# speedup vs baseline: 11.2551x; 11.2551x over previous
import functools

import jax
import jax.numpy as jnp
from jax import lax
from jax.experimental import pallas as pl
from jax.experimental.pallas import tpu as pltpu
from jax.experimental.pallas import tpu_sc as plsc

N = 10000
E = 320000
D_IN = 128
HIDDEN = 32
LATENT = 16
S = 1000

NC = 2
NS = 16
NW = NC * NS
L = 16

NP = 10240
ROWS_PT = NP // NS
EPT = E // NW
K = 80
NCHUNK = EPT // K
SP = 1024
SPT = SP // NW
AB = 8


def _sc_mesh():
    return plsc.VectorSubcoreMesh(
        core_axis_name="c", subcore_axis_name="s", num_cores=NC, num_subcores=NS
    )


_SC_PARAMS = pltpu.CompilerParams(use_tc_tiling_on_sc=False)


def _sc_degrees(src, dst):
    def body(src_hbm, dst_hbm, do_hbm, di_hbm, idx_v, ones_v, zrow_v, do_sh, di_sh):
        c = lax.axis_index("c")
        s = lax.axis_index("s")
        wid = c * NS + s
        zeros16 = jnp.zeros((L,), jnp.float32)
        ones16 = jnp.ones((L,), jnp.float32)
        for i in range(K // L):
            ones_v[pl.ds(i * L, L)] = ones16

        def zfill(i, carry):
            zrow_v[pl.ds(i * L, L)] = zeros16
            return carry

        lax.fori_loop(0, ROWS_PT // L, zfill, 0)
        r0 = s * ROWS_PT
        pltpu.sync_copy(zrow_v, do_sh.at[pl.ds(r0, ROWS_PT)])
        pltpu.sync_copy(zrow_v, di_sh.at[pl.ds(r0, ROWS_PT)])
        plsc.subcore_barrier()

        base = wid * EPT

        def chunk(i, carry):
            off = base + i * K
            pltpu.sync_copy(src_hbm.at[pl.ds(off, K)], idx_v)
            pltpu.sync_copy(ones_v, do_sh.at[idx_v], add=True)
            pltpu.sync_copy(dst_hbm.at[pl.ds(off, K)], idx_v)
            pltpu.sync_copy(ones_v, di_sh.at[idx_v], add=True)
            return carry

        lax.fori_loop(0, NCHUNK, chunk, 0)
        plsc.subcore_barrier()
        pltpu.sync_copy(do_sh.at[pl.ds(r0, ROWS_PT)], do_hbm.at[c, pl.ds(r0, ROWS_PT)])
        pltpu.sync_copy(di_sh.at[pl.ds(r0, ROWS_PT)], di_hbm.at[c, pl.ds(r0, ROWS_PT)])

    return pl.kernel(
        body,
        out_type=[
            jax.ShapeDtypeStruct((NC, NP), jnp.float32),
            jax.ShapeDtypeStruct((NC, NP), jnp.float32),
        ],
        mesh=_sc_mesh(),
        compiler_params=_SC_PARAMS,
        scratch_types=[
            pltpu.VMEM((K,), jnp.int32),
            pltpu.VMEM((K,), jnp.float32),
            pltpu.VMEM((ROWS_PT,), jnp.float32),
            pltpu.VMEM_SHARED((NP,), jnp.float32),
            pltpu.VMEM_SHARED((NP,), jnp.float32),
        ],
    )(src, dst)


def _sc_segsum(table, src, dst, W):
    def body(tab_hbm, src_hbm, dst_hbm, out_hbm, sidx_v, didx_v, rows_v, acc_sh, sem):
        c = lax.axis_index("c")
        s = lax.axis_index("s")
        wid = c * NS + s
        zeros16 = jnp.zeros((L,), jnp.float32)

        def zfill(i, carry):
            for j in range(W // L):
                rows_v[i, pl.ds(j * L, L)] = zeros16
            return carry

        lax.fori_loop(0, K, zfill, 0)
        r0 = s * ROWS_PT
        for j in range(ROWS_PT // K):
            pltpu.sync_copy(rows_v, acc_sh.at[pl.ds(r0 + j * K, K)])
        plsc.subcore_barrier()

        base = wid * EPT

        def chunk(i, carry):
            off = base + i * K
            pltpu.sync_copy(src_hbm.at[pl.ds(off, K)], sidx_v)
            pltpu.sync_copy(dst_hbm.at[pl.ds(off, K)], didx_v)
            pltpu.async_copy(tab_hbm.at[sidx_v], rows_v, sem).wait()
            pltpu.sync_copy(rows_v, acc_sh.at[didx_v], add=True)
            return carry

        lax.fori_loop(0, NCHUNK, chunk, 0)
        plsc.subcore_barrier()
        pltpu.sync_copy(acc_sh.at[pl.ds(r0, ROWS_PT)], out_hbm.at[c, pl.ds(r0, ROWS_PT)])

    return pl.kernel(
        body,
        out_type=jax.ShapeDtypeStruct((NC, NP, W), jnp.float32),
        mesh=_sc_mesh(),
        compiler_params=_SC_PARAMS,
        scratch_types=[
            pltpu.VMEM((K,), jnp.int32),
            pltpu.VMEM((K,), jnp.int32),
            pltpu.VMEM((K, W), jnp.float32),
            pltpu.VMEM_SHARED((NP, W), jnp.float32),
            pltpu.SemaphoreType.DMA,
        ],
    )(table, src, dst)


def _sc_sample(p0, p1, ab, samp):
    def body(p0_hbm, p1_hbm, ab_hbm, samp_hbm, g0_hbm, g1_hbm, gb_hbm,
             sidx_v, r0_v, r1_v, rb_v, sem):
        c = lax.axis_index("c")
        s = lax.axis_index("s")
        wid = c * NS + s
        base = wid * SPT
        pltpu.sync_copy(samp_hbm.at[pl.ds(base, SPT)], sidx_v)
        pltpu.async_copy(p0_hbm.at[sidx_v], r0_v, sem).wait()
        pltpu.async_copy(p1_hbm.at[sidx_v], r1_v, sem).wait()
        pltpu.async_copy(ab_hbm.at[sidx_v], rb_v, sem).wait()
        pltpu.sync_copy(r0_v, g0_hbm.at[pl.ds(base, SPT)])
        pltpu.sync_copy(r1_v, g1_hbm.at[pl.ds(base, SPT)])
        pltpu.sync_copy(rb_v, gb_hbm.at[pl.ds(base, SPT)])

    return pl.kernel(
        body,
        out_type=[
            jax.ShapeDtypeStruct((SP, LATENT), jnp.float32),
            jax.ShapeDtypeStruct((SP, LATENT), jnp.float32),
            jax.ShapeDtypeStruct((SP, AB), jnp.float32),
        ],
        mesh=_sc_mesh(),
        compiler_params=_SC_PARAMS,
        scratch_types=[
            pltpu.VMEM((SPT,), jnp.int32),
            pltpu.VMEM((SPT, LATENT), jnp.float32),
            pltpu.VMEM((SPT, LATENT), jnp.float32),
            pltpu.VMEM((SPT, AB), jnp.float32),
            pltpu.SemaphoreType.DMA,
        ],
    )(p0, p1, ab, samp)


def _tc_prep(x, W1, degt):
    BM = 2000

    def body(x_ref, w_ref, d_ref, h_ref, ab_ref):
        xw = jnp.dot(x_ref[...], w_ref[...], preferred_element_type=jnp.float32)
        dout = d_ref[:, 0:1] + d_ref[:, 1:2]
        din = d_ref[:, 2:3] + d_ref[:, 3:4]
        a = lax.rsqrt(jnp.maximum(dout, 1.0))
        b = lax.rsqrt(jnp.maximum(din, 1.0))
        h_ref[...] = xw * a
        ab_ref[...] = jnp.concatenate(
            [a, b, jnp.zeros((a.shape[0], AB - 2), jnp.float32)], axis=1)

    return pl.pallas_call(
        body,
        grid=(N // BM,),
        in_specs=[
            pl.BlockSpec((BM, D_IN), lambda i: (i, 0)),
            pl.BlockSpec((D_IN, HIDDEN), lambda i: (0, 0)),
            pl.BlockSpec((BM, 4), lambda i: (i, 0)),
        ],
        out_specs=[
            pl.BlockSpec((BM, HIDDEN), lambda i: (i, 0)),
            pl.BlockSpec((BM, AB), lambda i: (i, 0)),
        ],
        out_shape=[
            jax.ShapeDtypeStruct((N, HIDDEN), jnp.float32),
            jax.ShapeDtypeStruct((N, AB), jnp.float32),
        ],
    )(x, W1, degt)


def _tc_mid(p0, p1, ab, W2):
    BM = 2000

    def body(p0_ref, p1_ref, ab_ref, w_ref, o_ref):
        b = ab_ref[:, 1:2]
        a = ab_ref[:, 0:1]
        hidden = jnp.maximum((p0_ref[...] + p1_ref[...]) * b, 0.0)
        o_ref[...] = jnp.dot(hidden, w_ref[...], preferred_element_type=jnp.float32) * a

    return pl.pallas_call(
        body,
        grid=(N // BM,),
        in_specs=[
            pl.BlockSpec((BM, HIDDEN), lambda i: (i, 0)),
            pl.BlockSpec((BM, HIDDEN), lambda i: (i, 0)),
            pl.BlockSpec((BM, AB), lambda i: (i, 0)),
            pl.BlockSpec((HIDDEN, LATENT), lambda i: (0, 0)),
        ],
        out_specs=pl.BlockSpec((BM, LATENT), lambda i: (i, 0)),
        out_shape=jax.ShapeDtypeStruct((N, LATENT), jnp.float32),
    )(p0, p1, ab, W2)


def _tc_decoder(g0, g1, gb):
    R = 200

    def body(g0_ref, g1_ref, gb_ref, o_ref):
        i = pl.program_id(0)
        z = (g0_ref[...] + g1_ref[...]) * gb_ref[...]
        zr = (g0_ref[pl.ds(i * R, R), :] + g1_ref[pl.ds(i * R, R), :]) * gb_ref[pl.ds(i * R, R), :]
        gram = lax.dot_general(zr, z, (((1,), (1,)), ((), ())),
                               preferred_element_type=jnp.float32)
        zz = z * z
        sqc = lax.dot_general(jnp.ones((1, LATENT), jnp.float32), zz,
                              (((1,), (1,)), ((), ())),
                              preferred_element_type=jnp.float32)
        sqr = jnp.sum(zr * zr, axis=1, keepdims=True)
        d2 = jnp.maximum(sqr + sqc - 2.0 * gram, 0.0)
        o_ref[0] = gram
        o_ref[1] = jnp.sqrt(d2 + 1e-12)

    return pl.pallas_call(
        body,
        grid=(S // R,),
        in_specs=[
            pl.BlockSpec((S, LATENT), lambda i: (0, 0)),
            pl.BlockSpec((S, LATENT), lambda i: (0, 0)),
            pl.BlockSpec((S, 1), lambda i: (0, 0)),
        ],
        out_specs=pl.BlockSpec((2, R, S), lambda i: (0, i, 0)),
        out_shape=jax.ShapeDtypeStruct((2, S, S), jnp.float32),
    )(g0, g1, gb)


@jax.jit
def kernel(x, edge_index, sampled_nodes, W1, W2):
    src = edge_index[0]
    dst = edge_index[1]
    deg_out, deg_in = _sc_degrees(src, dst)
    degt = jnp.concatenate([deg_out[:, :N], deg_in[:, :N]], axis=0).T
    h1p, ab = _tc_prep(x, W1, degt)
    s1p = _sc_segsum(h1p, src, dst, HIDDEN)
    h2p = _tc_mid(s1p[0, :N], s1p[1, :N], ab, W2)
    s2p = _sc_segsum(h2p, src, dst, LATENT)
    samp = jnp.concatenate([sampled_nodes, jnp.zeros((SP - S,), jnp.int32)])
    g0, g1, gab = _sc_sample(s2p[0], s2p[1], ab, samp)
    out = _tc_decoder(g0[:S], g1[:S], gab[:S, 1:2])
    return out.reshape(2, S * S)


# speedup vs baseline: 33.2617x; 2.9552x over previous
import functools

import jax
import jax.numpy as jnp
from jax import lax
from jax.experimental import pallas as pl
from jax.experimental.pallas import tpu as pltpu
from jax.experimental.pallas import tpu_sc as plsc

N = 10000
E = 320000
D_IN = 128
HIDDEN = 32
LATENT = 16
S = 1000

NC = 2
NS = 16
NW = NC * NS
L = 16

NP = 10240
ROWS_PT = NP // NS
EPT = E // NW
K = 80
NCHUNK = EPT // K
SP = 1024
SPT = SP // NW
AB = 8


def _sc_mesh():
    return plsc.VectorSubcoreMesh(
        core_axis_name="c", subcore_axis_name="s", num_cores=NC, num_subcores=NS
    )


_SC_PARAMS = pltpu.CompilerParams(use_tc_tiling_on_sc=False)


def _sc_degrees(src2d, dst2d):
    NB = 5
    NT = NCHUNK // NB

    def body(src_hbm, dst_hbm, do_hbm, di_hbm, sidx_v, didx_v, ones_v, zrow_v,
             do_sh, di_sh, *sems):
        sa = sems[:NB]
        sb = sems[NB:]
        c = lax.axis_index("c")
        s = lax.axis_index("s")
        wid = c * NS + s
        zeros16 = jnp.zeros((L,), jnp.float32)
        ones16 = jnp.ones((L,), jnp.float32)
        for i in range(K // L):
            ones_v[pl.ds(i * L, L)] = ones16

        def zfill(i, carry):
            zrow_v[pl.ds(i * L, L)] = zeros16
            return carry

        lax.fori_loop(0, ROWS_PT // L, zfill, 0)
        r0 = s * ROWS_PT
        pltpu.sync_copy(zrow_v, do_sh.at[pl.ds(r0, ROWS_PT)])
        pltpu.sync_copy(zrow_v, di_sh.at[pl.ds(r0, ROWS_PT)])
        crow = wid * NCHUNK
        pltpu.sync_copy(src_hbm.at[pl.ds(crow, NCHUNK)], sidx_v)
        pltpu.sync_copy(dst_hbm.at[pl.ds(crow, NCHUNK)], didx_v)
        plsc.subcore_barrier()

        def outer(t, carry):
            for i in range(NB):
                j = t * NB + i

                @pl.when(t > 0)
                def _():
                    pltpu.make_async_copy(ones_v, do_sh.at[sidx_v.at[j]], sa[i]).wait()
                    pltpu.make_async_copy(ones_v, di_sh.at[didx_v.at[j]], sb[i]).wait()

                pltpu.async_copy(ones_v, do_sh.at[sidx_v.at[j]], sa[i], add=True)
                pltpu.async_copy(ones_v, di_sh.at[didx_v.at[j]], sb[i], add=True)
            return carry

        lax.fori_loop(0, NT, outer, 0)
        for i in range(NB):
            pltpu.make_async_copy(ones_v, do_sh.at[sidx_v.at[i]], sa[i]).wait()
            pltpu.make_async_copy(ones_v, di_sh.at[didx_v.at[i]], sb[i]).wait()
        plsc.subcore_barrier()
        pltpu.sync_copy(do_sh.at[pl.ds(r0, ROWS_PT)], do_hbm.at[c, pl.ds(r0, ROWS_PT)])
        pltpu.sync_copy(di_sh.at[pl.ds(r0, ROWS_PT)], di_hbm.at[c, pl.ds(r0, ROWS_PT)])

    return pl.kernel(
        body,
        out_type=[
            jax.ShapeDtypeStruct((NC, NP), jnp.float32),
            jax.ShapeDtypeStruct((NC, NP), jnp.float32),
        ],
        mesh=_sc_mesh(),
        compiler_params=_SC_PARAMS,
        scratch_types=[
            pltpu.VMEM((NCHUNK, K), jnp.int32),
            pltpu.VMEM((NCHUNK, K), jnp.int32),
            pltpu.VMEM((K,), jnp.float32),
            pltpu.VMEM((ROWS_PT,), jnp.float32),
            pltpu.VMEM_SHARED((NP,), jnp.float32),
            pltpu.VMEM_SHARED((NP,), jnp.float32),
        ] + [pltpu.SemaphoreType.DMA] * 10,
    )(src2d, dst2d)


def _sc_segsum(table, src2d, dst2d, W):
    NB = 5
    G = 3
    NT = NCHUNK // NB

    def body(tab_hbm, src_hbm, dst_hbm, out_hbm, sidx_v, didx_v, rows_v, acc_sh, *sems):
        c = lax.axis_index("c")
        s = lax.axis_index("s")
        wid = c * NS + s
        zeros16 = jnp.zeros((L,), jnp.float32)

        def zfill(i, carry):
            for j in range(W // L):
                rows_v[0, i, pl.ds(j * L, L)] = zeros16
            return carry

        lax.fori_loop(0, K, zfill, 0)
        r0 = s * ROWS_PT
        for j in range(ROWS_PT // K):
            pltpu.sync_copy(rows_v.at[0], acc_sh.at[pl.ds(r0 + j * K, K)])
        crow = wid * NCHUNK
        pltpu.sync_copy(src_hbm.at[pl.ds(crow, NCHUNK)], sidx_v)
        pltpu.sync_copy(dst_hbm.at[pl.ds(crow, NCHUNK)], didx_v)
        plsc.subcore_barrier()

        for j in range(G):
            pltpu.async_copy(tab_hbm.at[sidx_v.at[j]], rows_v.at[j], sems[j])

        def outer(t, carry):
            for i in range(NB):
                j = t * NB + i
                pltpu.make_async_copy(
                    tab_hbm.at[sidx_v.at[j]], rows_v.at[i], sems[i]).wait()
                pltpu.sync_copy(rows_v.at[i], acc_sh.at[didx_v.at[j]], add=True)
                jj = j + G
                bg = (i + G) % NB

                @pl.when(jj < NCHUNK)
                def _():
                    pltpu.async_copy(tab_hbm.at[sidx_v.at[jj]], rows_v.at[bg], sems[bg])
            return carry

        lax.fori_loop(0, NT, outer, 0)
        plsc.subcore_barrier()
        pltpu.sync_copy(acc_sh.at[pl.ds(r0, ROWS_PT)], out_hbm.at[c, pl.ds(r0, ROWS_PT)])

    return pl.kernel(
        body,
        out_type=jax.ShapeDtypeStruct((NC, NP, W), jnp.float32),
        mesh=_sc_mesh(),
        compiler_params=_SC_PARAMS,
        scratch_types=[
            pltpu.VMEM((NCHUNK, K), jnp.int32),
            pltpu.VMEM((NCHUNK, K), jnp.int32),
            pltpu.VMEM((NB, K, W), jnp.float32),
            pltpu.VMEM_SHARED((NP, W), jnp.float32),
        ] + [pltpu.SemaphoreType.DMA] * NB,
    )(table, src2d, dst2d)


def _sc_sample(p0, p1, ab, samp):
    def body(p0_hbm, p1_hbm, ab_hbm, samp_hbm, g0_hbm, g1_hbm, gb_hbm,
             sidx_v, r0_v, r1_v, rb_v, sem):
        c = lax.axis_index("c")
        s = lax.axis_index("s")
        wid = c * NS + s
        base = wid * SPT
        pltpu.sync_copy(samp_hbm.at[pl.ds(base, SPT)], sidx_v)
        pltpu.async_copy(p0_hbm.at[sidx_v], r0_v, sem).wait()
        pltpu.async_copy(p1_hbm.at[sidx_v], r1_v, sem).wait()
        pltpu.async_copy(ab_hbm.at[sidx_v], rb_v, sem).wait()
        pltpu.sync_copy(r0_v, g0_hbm.at[pl.ds(base, SPT)])
        pltpu.sync_copy(r1_v, g1_hbm.at[pl.ds(base, SPT)])
        pltpu.sync_copy(rb_v, gb_hbm.at[pl.ds(base, SPT)])

    return pl.kernel(
        body,
        out_type=[
            jax.ShapeDtypeStruct((SP, LATENT), jnp.float32),
            jax.ShapeDtypeStruct((SP, LATENT), jnp.float32),
            jax.ShapeDtypeStruct((SP, AB), jnp.float32),
        ],
        mesh=_sc_mesh(),
        compiler_params=_SC_PARAMS,
        scratch_types=[
            pltpu.VMEM((SPT,), jnp.int32),
            pltpu.VMEM((SPT, LATENT), jnp.float32),
            pltpu.VMEM((SPT, LATENT), jnp.float32),
            pltpu.VMEM((SPT, AB), jnp.float32),
            pltpu.SemaphoreType.DMA,
        ],
    )(p0, p1, ab, samp)


def _tc_prep(x, W1, degt):
    BM = 2000

    def body(x_ref, w_ref, d_ref, h_ref, ab_ref):
        xw = jnp.dot(x_ref[...], w_ref[...], preferred_element_type=jnp.float32)
        dout = d_ref[:, 0:1] + d_ref[:, 1:2]
        din = d_ref[:, 2:3] + d_ref[:, 3:4]
        a = lax.rsqrt(jnp.maximum(dout, 1.0))
        b = lax.rsqrt(jnp.maximum(din, 1.0))
        h_ref[...] = xw * a
        ab_ref[...] = jnp.concatenate(
            [a, b, jnp.zeros((a.shape[0], AB - 2), jnp.float32)], axis=1)

    return pl.pallas_call(
        body,
        grid=(N // BM,),
        in_specs=[
            pl.BlockSpec((BM, D_IN), lambda i: (i, 0)),
            pl.BlockSpec((D_IN, HIDDEN), lambda i: (0, 0)),
            pl.BlockSpec((BM, 4), lambda i: (i, 0)),
        ],
        out_specs=[
            pl.BlockSpec((BM, HIDDEN), lambda i: (i, 0)),
            pl.BlockSpec((BM, AB), lambda i: (i, 0)),
        ],
        out_shape=[
            jax.ShapeDtypeStruct((N, HIDDEN), jnp.float32),
            jax.ShapeDtypeStruct((N, AB), jnp.float32),
        ],
    )(x, W1, degt)


def _tc_mid(p0, p1, ab, W2):
    BM = 2000

    def body(p0_ref, p1_ref, ab_ref, w_ref, o_ref):
        b = ab_ref[:, 1:2]
        a = ab_ref[:, 0:1]
        hidden = jnp.maximum((p0_ref[...] + p1_ref[...]) * b, 0.0)
        o_ref[...] = jnp.dot(hidden, w_ref[...], preferred_element_type=jnp.float32) * a

    return pl.pallas_call(
        body,
        grid=(N // BM,),
        in_specs=[
            pl.BlockSpec((BM, HIDDEN), lambda i: (i, 0)),
            pl.BlockSpec((BM, HIDDEN), lambda i: (i, 0)),
            pl.BlockSpec((BM, AB), lambda i: (i, 0)),
            pl.BlockSpec((HIDDEN, LATENT), lambda i: (0, 0)),
        ],
        out_specs=pl.BlockSpec((BM, LATENT), lambda i: (i, 0)),
        out_shape=jax.ShapeDtypeStruct((N, LATENT), jnp.float32),
    )(p0, p1, ab, W2)


def _tc_decoder(g0, g1, gb):
    R = 200

    def body(g0_ref, g1_ref, gb_ref, o_ref):
        i = pl.program_id(0)
        z = (g0_ref[...] + g1_ref[...]) * gb_ref[...]
        zr = (g0_ref[pl.ds(i * R, R), :] + g1_ref[pl.ds(i * R, R), :]) * gb_ref[pl.ds(i * R, R), :]
        gram = lax.dot_general(zr, z, (((1,), (1,)), ((), ())),
                               preferred_element_type=jnp.float32)
        zz = z * z
        sqc = lax.dot_general(jnp.ones((1, LATENT), jnp.float32), zz,
                              (((1,), (1,)), ((), ())),
                              preferred_element_type=jnp.float32)
        sqr = jnp.sum(zr * zr, axis=1, keepdims=True)
        d2 = jnp.maximum(sqr + sqc - 2.0 * gram, 0.0)
        o_ref[0] = gram
        o_ref[1] = jnp.sqrt(d2 + 1e-12)

    return pl.pallas_call(
        body,
        grid=(S // R,),
        in_specs=[
            pl.BlockSpec((S, LATENT), lambda i: (0, 0)),
            pl.BlockSpec((S, LATENT), lambda i: (0, 0)),
            pl.BlockSpec((S, 1), lambda i: (0, 0)),
        ],
        out_specs=pl.BlockSpec((2, R, S), lambda i: (0, i, 0)),
        out_shape=jax.ShapeDtypeStruct((2, S, S), jnp.float32),
    )(g0, g1, gb)


@jax.jit
def kernel(x, edge_index, sampled_nodes, W1, W2):
    src2d = edge_index[0].reshape(E // K, K)
    dst2d = edge_index[1].reshape(E // K, K)
    deg_out, deg_in = _sc_degrees(src2d, dst2d)
    degt = jnp.concatenate([deg_out[:, :N], deg_in[:, :N]], axis=0).T
    h1p, ab = _tc_prep(x, W1, degt)
    s1p = _sc_segsum(h1p, src2d, dst2d, HIDDEN)
    h2p = _tc_mid(s1p[0, :N], s1p[1, :N], ab, W2)
    s2p = _sc_segsum(h2p, src2d, dst2d, LATENT)
    samp = jnp.concatenate([sampled_nodes, jnp.zeros((SP - S,), jnp.int32)])
    g0, g1, gab = _sc_sample(s2p[0], s2p[1], ab, samp)
    out = _tc_decoder(g0[:S], g1[:S], gab[:S, 1:2])
    return out.reshape(2, S * S)


# speedup vs baseline: 39.2192x; 1.1791x over previous
import functools

import jax
import jax.numpy as jnp
from jax import lax
from jax.experimental import pallas as pl
from jax.experimental.pallas import tpu as pltpu
from jax.experimental.pallas import tpu_sc as plsc

N = 10000
E = 320000
D_IN = 128
HIDDEN = 32
LATENT = 16
S = 1000

NC = 2
NS = 16
NW = NC * NS
L = 16

NP = 10240
ROWS_PT = NP // NS
EPT = E // NW
K = 80
NCHUNK = EPT // K
SP = 1024
SPT = SP // NW
AB = 8


def _sc_mesh():
    return plsc.VectorSubcoreMesh(
        core_axis_name="c", subcore_axis_name="s", num_cores=NC, num_subcores=NS
    )


_SC_PARAMS = pltpu.CompilerParams(use_tc_tiling_on_sc=False)


def _sc_degrees(src2d, dst2d):
    NB = 5
    NT = NCHUNK // NB

    def body(src_hbm, dst_hbm, do_hbm, di_hbm, sidx_v, didx_v, ones_v, zrow_v,
             do_sh, di_sh, *sems):
        sa = sems[:NB]
        sb = sems[NB:]
        c = lax.axis_index("c")
        s = lax.axis_index("s")
        wid = c * NS + s
        zeros16 = jnp.zeros((L,), jnp.float32)
        ones16 = jnp.ones((L,), jnp.float32)
        for i in range(K // L):
            ones_v[pl.ds(i * L, L)] = ones16

        def zfill(i, carry):
            zrow_v[pl.ds(i * L, L)] = zeros16
            return carry

        lax.fori_loop(0, ROWS_PT // L, zfill, 0)
        r0 = s * ROWS_PT
        pltpu.sync_copy(zrow_v, do_sh.at[pl.ds(r0, ROWS_PT)])
        pltpu.sync_copy(zrow_v, di_sh.at[pl.ds(r0, ROWS_PT)])
        crow = wid * NCHUNK
        pltpu.sync_copy(src_hbm.at[pl.ds(crow, NCHUNK)], sidx_v)
        pltpu.sync_copy(dst_hbm.at[pl.ds(crow, NCHUNK)], didx_v)
        plsc.subcore_barrier()

        def outer(t, carry):
            for i in range(NB):
                j = t * NB + i

                @pl.when(t > 0)
                def _():
                    pltpu.make_async_copy(ones_v, do_sh.at[sidx_v.at[j]], sa[i]).wait()
                    pltpu.make_async_copy(ones_v, di_sh.at[didx_v.at[j]], sb[i]).wait()

                pltpu.async_copy(ones_v, do_sh.at[sidx_v.at[j]], sa[i], add=True)
                pltpu.async_copy(ones_v, di_sh.at[didx_v.at[j]], sb[i], add=True)
            return carry

        lax.fori_loop(0, NT, outer, 0)
        for i in range(NB):
            pltpu.make_async_copy(ones_v, do_sh.at[sidx_v.at[i]], sa[i]).wait()
            pltpu.make_async_copy(ones_v, di_sh.at[didx_v.at[i]], sb[i]).wait()
        plsc.subcore_barrier()
        pltpu.sync_copy(do_sh.at[pl.ds(r0, ROWS_PT)], do_hbm.at[c, pl.ds(r0, ROWS_PT)])
        pltpu.sync_copy(di_sh.at[pl.ds(r0, ROWS_PT)], di_hbm.at[c, pl.ds(r0, ROWS_PT)])

    return pl.kernel(
        body,
        out_type=[
            jax.ShapeDtypeStruct((NC, NP), jnp.float32),
            jax.ShapeDtypeStruct((NC, NP), jnp.float32),
        ],
        mesh=_sc_mesh(),
        compiler_params=_SC_PARAMS,
        scratch_types=[
            pltpu.VMEM((NCHUNK, K), jnp.int32),
            pltpu.VMEM((NCHUNK, K), jnp.int32),
            pltpu.VMEM((K,), jnp.float32),
            pltpu.VMEM((ROWS_PT,), jnp.float32),
            pltpu.VMEM_SHARED((NP,), jnp.float32),
            pltpu.VMEM_SHARED((NP,), jnp.float32),
        ] + [pltpu.SemaphoreType.DMA] * 10,
    )(src2d, dst2d)


def _sc_segsum(table, src2d, dst2d, W):
    NB = 5
    G = 3
    NT = NCHUNK // NB

    def body(tab_hbm, src_hbm, dst_hbm, out_hbm, sidx_v, didx_v, rows_v, acc_sh, *sems):
        sg = sems[:NB]
        ss = sems[NB:]
        c = lax.axis_index("c")
        s = lax.axis_index("s")
        wid = c * NS + s
        zeros16 = jnp.zeros((L,), jnp.float32)

        def zfill(i, carry):
            for j in range(W // L):
                rows_v[0, i, pl.ds(j * L, L)] = zeros16
            return carry

        lax.fori_loop(0, K, zfill, 0)
        r0 = s * ROWS_PT
        for j in range(ROWS_PT // K):
            pltpu.sync_copy(rows_v.at[0], acc_sh.at[pl.ds(r0 + j * K, K)])
        crow = wid * NCHUNK
        pltpu.async_copy(src_hbm.at[pl.ds(crow, NCHUNK)], sidx_v, sg[0])
        pltpu.async_copy(dst_hbm.at[pl.ds(crow, NCHUNK)], didx_v, sg[1])
        pltpu.make_async_copy(src_hbm.at[pl.ds(crow, NCHUNK)], sidx_v, sg[0]).wait()
        pltpu.make_async_copy(dst_hbm.at[pl.ds(crow, NCHUNK)], didx_v, sg[1]).wait()
        plsc.subcore_barrier()

        for j in range(G):
            pltpu.async_copy(tab_hbm.at[sidx_v.at[j]], rows_v.at[j], sg[j])

        def outer(t, carry):
            for i in range(NB):
                j = t * NB + i
                pltpu.make_async_copy(
                    tab_hbm.at[sidx_v.at[j]], rows_v.at[i], sg[i]).wait()
                pltpu.async_copy(rows_v.at[i], acc_sh.at[didx_v.at[j]], ss[i], add=True)
                jj = j + G
                bg = (i + G) % NB

                def start_gather():
                    pltpu.make_async_copy(
                        rows_v.at[bg], acc_sh.at[didx_v.at[j]], ss[bg]).wait()
                    pltpu.async_copy(tab_hbm.at[sidx_v.at[jj]], rows_v.at[bg], sg[bg])

                if i < NB - G:
                    @pl.when(t > 0)
                    def _():
                        start_gather()

                    @pl.when(jnp.logical_and(t == 0, jj < NCHUNK))
                    def _():
                        pltpu.async_copy(tab_hbm.at[sidx_v.at[jj]], rows_v.at[bg], sg[bg])
                else:
                    @pl.when(jj < NCHUNK)
                    def _():
                        start_gather()
            return carry

        lax.fori_loop(0, NT, outer, 0)
        for i in range(NB):
            pltpu.make_async_copy(rows_v.at[i], acc_sh.at[didx_v.at[i]], ss[i]).wait()
        plsc.subcore_barrier()
        pltpu.sync_copy(acc_sh.at[pl.ds(r0, ROWS_PT)], out_hbm.at[c, pl.ds(r0, ROWS_PT)])

    return pl.kernel(
        body,
        out_type=jax.ShapeDtypeStruct((NC, NP, W), jnp.float32),
        mesh=_sc_mesh(),
        compiler_params=_SC_PARAMS,
        scratch_types=[
            pltpu.VMEM((NCHUNK, K), jnp.int32),
            pltpu.VMEM((NCHUNK, K), jnp.int32),
            pltpu.VMEM((NB, K, W), jnp.float32),
            pltpu.VMEM_SHARED((NP, W), jnp.float32),
        ] + [pltpu.SemaphoreType.DMA] * (2 * NB),
    )(table, src2d, dst2d)


def _sc_segsum_sample(table, src2d, dst2d, ab, samp):
    W = LATENT
    NB = 5
    G = 3
    NT = NCHUNK // NB
    SGT = SP // NS

    def body(tab_hbm, src_hbm, dst_hbm, ab_hbm, samp_hbm, g_hbm, gab_hbm,
             sidx_v, didx_v, rows_v, smp_v, gr_v, gab_v, acc_sh, *sems):
        sg = sems[:NB]
        ss = sems[NB:]
        c = lax.axis_index("c")
        s = lax.axis_index("s")
        wid = c * NS + s
        zeros16 = jnp.zeros((L,), jnp.float32)

        def zfill(i, carry):
            for j in range(W // L):
                rows_v[0, i, pl.ds(j * L, L)] = zeros16
            return carry

        lax.fori_loop(0, K, zfill, 0)
        r0 = s * ROWS_PT
        for j in range(ROWS_PT // K):
            pltpu.sync_copy(rows_v.at[0], acc_sh.at[pl.ds(r0 + j * K, K)])
        crow = wid * NCHUNK
        pltpu.async_copy(src_hbm.at[pl.ds(crow, NCHUNK)], sidx_v, sg[0])
        pltpu.async_copy(dst_hbm.at[pl.ds(crow, NCHUNK)], didx_v, sg[1])
        pltpu.sync_copy(samp_hbm, smp_v)
        pltpu.make_async_copy(src_hbm.at[pl.ds(crow, NCHUNK)], sidx_v, sg[0]).wait()
        pltpu.make_async_copy(dst_hbm.at[pl.ds(crow, NCHUNK)], didx_v, sg[1]).wait()
        plsc.subcore_barrier()

        for j in range(G):
            pltpu.async_copy(tab_hbm.at[sidx_v.at[j]], rows_v.at[j], sg[j])

        def outer(t, carry):
            for i in range(NB):
                j = t * NB + i
                pltpu.make_async_copy(
                    tab_hbm.at[sidx_v.at[j]], rows_v.at[i], sg[i]).wait()
                pltpu.async_copy(rows_v.at[i], acc_sh.at[didx_v.at[j]], ss[i], add=True)
                jj = j + G
                bg = (i + G) % NB

                def start_gather():
                    pltpu.make_async_copy(
                        rows_v.at[bg], acc_sh.at[didx_v.at[j]], ss[bg]).wait()
                    pltpu.async_copy(tab_hbm.at[sidx_v.at[jj]], rows_v.at[bg], sg[bg])

                if i < NB - G:
                    @pl.when(t > 0)
                    def _():
                        start_gather()

                    @pl.when(jnp.logical_and(t == 0, jj < NCHUNK))
                    def _():
                        pltpu.async_copy(tab_hbm.at[sidx_v.at[jj]], rows_v.at[bg], sg[bg])
                else:
                    @pl.when(jj < NCHUNK)
                    def _():
                        start_gather()
            return carry

        lax.fori_loop(0, NT, outer, 0)
        for i in range(NB):
            pltpu.make_async_copy(rows_v.at[i], acc_sh.at[didx_v.at[i]], ss[i]).wait()
        plsc.subcore_barrier()
        srow = s * SGT
        pltpu.async_copy(acc_sh.at[smp_v.at[pl.ds(srow, SGT)]], gr_v, sg[0])
        abase = wid * SPT
        pltpu.async_copy(ab_hbm.at[smp_v.at[pl.ds(abase, SPT)]], gab_v, sg[1])
        pltpu.make_async_copy(acc_sh.at[smp_v.at[pl.ds(srow, SGT)]], gr_v, sg[0]).wait()
        pltpu.make_async_copy(ab_hbm.at[smp_v.at[pl.ds(abase, SPT)]], gab_v, sg[1]).wait()
        pltpu.sync_copy(gr_v, g_hbm.at[c, pl.ds(srow, SGT)])
        pltpu.sync_copy(gab_v, gab_hbm.at[pl.ds(abase, SPT)])

    return pl.kernel(
        body,
        out_type=[
            jax.ShapeDtypeStruct((NC, SP, LATENT), jnp.float32),
            jax.ShapeDtypeStruct((SP, AB), jnp.float32),
        ],
        mesh=_sc_mesh(),
        compiler_params=_SC_PARAMS,
        scratch_types=[
            pltpu.VMEM((NCHUNK, K), jnp.int32),
            pltpu.VMEM((NCHUNK, K), jnp.int32),
            pltpu.VMEM((NB, K, W), jnp.float32),
            pltpu.VMEM((SP,), jnp.int32),
            pltpu.VMEM((SP // NS, LATENT), jnp.float32),
            pltpu.VMEM((SPT, AB), jnp.float32),
            pltpu.VMEM_SHARED((NP, W), jnp.float32),
        ] + [pltpu.SemaphoreType.DMA] * 10,
    )(table, src2d, dst2d, ab, samp)


def _tc_prep(x, W1, degt):
    BM = 2000

    def body(x_ref, w_ref, d_ref, h_ref, ab_ref):
        xw = jnp.dot(x_ref[...], w_ref[...], preferred_element_type=jnp.float32)
        dout = d_ref[:, 0:1] + d_ref[:, 1:2]
        din = d_ref[:, 2:3] + d_ref[:, 3:4]
        a = lax.rsqrt(jnp.maximum(dout, 1.0))
        b = lax.rsqrt(jnp.maximum(din, 1.0))
        h_ref[...] = xw * a
        ab_ref[...] = jnp.concatenate(
            [a, b, jnp.zeros((a.shape[0], AB - 2), jnp.float32)], axis=1)

    return pl.pallas_call(
        body,
        grid=(N // BM,),
        in_specs=[
            pl.BlockSpec((BM, D_IN), lambda i: (i, 0)),
            pl.BlockSpec((D_IN, HIDDEN), lambda i: (0, 0)),
            pl.BlockSpec((BM, 4), lambda i: (i, 0)),
        ],
        out_specs=[
            pl.BlockSpec((BM, HIDDEN), lambda i: (i, 0)),
            pl.BlockSpec((BM, AB), lambda i: (i, 0)),
        ],
        out_shape=[
            jax.ShapeDtypeStruct((N, HIDDEN), jnp.float32),
            jax.ShapeDtypeStruct((N, AB), jnp.float32),
        ],
    )(x, W1, degt)


def _tc_mid(p0, p1, ab, W2):
    BM = 2000

    def body(p0_ref, p1_ref, ab_ref, w_ref, o_ref):
        b = ab_ref[:, 1:2]
        a = ab_ref[:, 0:1]
        hidden = jnp.maximum((p0_ref[...] + p1_ref[...]) * b, 0.0)
        o_ref[...] = jnp.dot(hidden, w_ref[...], preferred_element_type=jnp.float32) * a

    return pl.pallas_call(
        body,
        grid=(N // BM,),
        in_specs=[
            pl.BlockSpec((BM, HIDDEN), lambda i: (i, 0)),
            pl.BlockSpec((BM, HIDDEN), lambda i: (i, 0)),
            pl.BlockSpec((BM, AB), lambda i: (i, 0)),
            pl.BlockSpec((HIDDEN, LATENT), lambda i: (0, 0)),
        ],
        out_specs=pl.BlockSpec((BM, LATENT), lambda i: (i, 0)),
        out_shape=jax.ShapeDtypeStruct((N, LATENT), jnp.float32),
    )(p0, p1, ab, W2)


def _tc_decoder(g0, g1, gb):
    R = 200

    def body(g0_ref, g1_ref, gb_ref, o_ref):
        i = pl.program_id(0)
        z = (g0_ref[...] + g1_ref[...]) * gb_ref[...]
        zr = (g0_ref[pl.ds(i * R, R), :] + g1_ref[pl.ds(i * R, R), :]) * gb_ref[pl.ds(i * R, R), :]
        gram = lax.dot_general(zr, z, (((1,), (1,)), ((), ())),
                               preferred_element_type=jnp.float32)
        zz = z * z
        sqc = lax.dot_general(jnp.ones((1, LATENT), jnp.float32), zz,
                              (((1,), (1,)), ((), ())),
                              preferred_element_type=jnp.float32)
        sqr = jnp.sum(zr * zr, axis=1, keepdims=True)
        d2 = jnp.maximum(sqr + sqc - 2.0 * gram, 0.0)
        o_ref[0] = gram
        o_ref[1] = jnp.sqrt(d2 + 1e-12)

    return pl.pallas_call(
        body,
        grid=(S // R,),
        in_specs=[
            pl.BlockSpec((S, LATENT), lambda i: (0, 0)),
            pl.BlockSpec((S, LATENT), lambda i: (0, 0)),
            pl.BlockSpec((S, 1), lambda i: (0, 0)),
        ],
        out_specs=pl.BlockSpec((2, R, S), lambda i: (0, i, 0)),
        out_shape=jax.ShapeDtypeStruct((2, S, S), jnp.float32),
    )(g0, g1, gb)


@jax.jit
def kernel(x, edge_index, sampled_nodes, W1, W2):
    src2d = edge_index[0].reshape(E // K, K)
    dst2d = edge_index[1].reshape(E // K, K)
    deg_out, deg_in = _sc_degrees(src2d, dst2d)
    degt = jnp.concatenate([deg_out[:, :N], deg_in[:, :N]], axis=0).T
    h1p, ab = _tc_prep(x, W1, degt)
    s1p = _sc_segsum(h1p, src2d, dst2d, HIDDEN)
    h2p = _tc_mid(s1p[0, :N], s1p[1, :N], ab, W2)
    samp = jnp.concatenate([sampled_nodes, jnp.zeros((SP - S,), jnp.int32)])
    g, gab = _sc_segsum_sample(h2p, src2d, dst2d, ab, samp)
    out = _tc_decoder(g[0, :S], g[1, :S], gab[:S, 1:2])
    return out.reshape(2, S * S)


# speedup vs baseline: 43.2231x; 1.1021x over previous
import jax
import jax.numpy as jnp
from jax import lax
from jax.experimental import pallas as pl
from jax.experimental.pallas import tpu as pltpu
from jax.experimental.pallas import tpu_sc as plsc

N = 10000
E = 320000
D_IN = 128
HIDDEN = 32
LATENT = 16
S = 1000

NC = 2
NS = 16
NW = NC * NS
L = 16

NP = 10240
ROWS_PT = NP // NS
K = 128
R2D = E // K
NCHUNK = 78
NTAIL = R2D - NW * NCHUNK
NB = 6
G = 3
NT = NCHUNK // NB
SP = 1024
SPT = SP // NW
AB = 8


def _sc_mesh():
    return plsc.VectorSubcoreMesh(
        core_axis_name="c", subcore_axis_name="s", num_cores=NC, num_subcores=NS
    )


_SC_PARAMS = pltpu.CompilerParams(use_tc_tiling_on_sc=False)


def _sc_degrees(src2d, dst2d):
    def body(src_hbm, dst_hbm, do_hbm, di_hbm, sidx_v, didx_v, tsx_v, tdx_v,
             ones_v, zrow_v, do_sh, di_sh, *sems):
        sa = sems[:NB]
        sb = sems[NB:]
        c = lax.axis_index("c")
        s = lax.axis_index("s")
        wid = c * NS + s
        zeros16 = jnp.zeros((L,), jnp.float32)
        ones16 = jnp.ones((L,), jnp.float32)
        for i in range(K // L):
            ones_v[pl.ds(i * L, L)] = ones16

        def zfill(i, carry):
            zrow_v[pl.ds(i * L, L)] = zeros16
            return carry

        lax.fori_loop(0, ROWS_PT // L, zfill, 0)
        r0 = s * ROWS_PT
        pltpu.sync_copy(zrow_v, do_sh.at[pl.ds(r0, ROWS_PT)])
        pltpu.sync_copy(zrow_v, di_sh.at[pl.ds(r0, ROWS_PT)])
        crow = wid * NCHUNK
        pltpu.sync_copy(src_hbm.at[pl.ds(crow, NCHUNK)], sidx_v)
        pltpu.sync_copy(dst_hbm.at[pl.ds(crow, NCHUNK)], didx_v)

        @pl.when(wid < NTAIL)
        def _():
            trow = NW * NCHUNK + wid
            pltpu.sync_copy(src_hbm.at[trow], tsx_v)
            pltpu.sync_copy(dst_hbm.at[trow], tdx_v)

        plsc.subcore_barrier()

        def outer(t, carry):
            for i in range(NB):
                j = t * NB + i

                @pl.when(t > 0)
                def _():
                    pltpu.make_async_copy(ones_v, do_sh.at[sidx_v.at[j]], sa[i]).wait()
                    pltpu.make_async_copy(ones_v, di_sh.at[didx_v.at[j]], sb[i]).wait()

                pltpu.async_copy(ones_v, do_sh.at[sidx_v.at[j]], sa[i], add=True)
                pltpu.async_copy(ones_v, di_sh.at[didx_v.at[j]], sb[i], add=True)
            return carry

        lax.fori_loop(0, NT, outer, 0)
        for i in range(NB):
            pltpu.make_async_copy(ones_v, do_sh.at[sidx_v.at[i]], sa[i]).wait()
            pltpu.make_async_copy(ones_v, di_sh.at[didx_v.at[i]], sb[i]).wait()

        @pl.when(wid < NTAIL)
        def _():
            pltpu.sync_copy(ones_v, do_sh.at[tsx_v], add=True)
            pltpu.sync_copy(ones_v, di_sh.at[tdx_v], add=True)

        plsc.subcore_barrier()
        pltpu.sync_copy(do_sh.at[pl.ds(r0, ROWS_PT)], do_hbm.at[c, pl.ds(r0, ROWS_PT)])
        pltpu.sync_copy(di_sh.at[pl.ds(r0, ROWS_PT)], di_hbm.at[c, pl.ds(r0, ROWS_PT)])

    return pl.kernel(
        body,
        out_type=[
            jax.ShapeDtypeStruct((NC, NP), jnp.float32),
            jax.ShapeDtypeStruct((NC, NP), jnp.float32),
        ],
        mesh=_sc_mesh(),
        compiler_params=_SC_PARAMS,
        scratch_types=[
            pltpu.VMEM((NCHUNK, K), jnp.int32),
            pltpu.VMEM((NCHUNK, K), jnp.int32),
            pltpu.VMEM((K,), jnp.int32),
            pltpu.VMEM((K,), jnp.int32),
            pltpu.VMEM((K,), jnp.float32),
            pltpu.VMEM((ROWS_PT,), jnp.float32),
            pltpu.VMEM_SHARED((NP,), jnp.float32),
            pltpu.VMEM_SHARED((NP,), jnp.float32),
        ] + [pltpu.SemaphoreType.DMA] * (2 * NB),
    )(src2d, dst2d)


def _segsum_core(tab_hbm, sidx_v, didx_v, tsx_v, tdx_v, rows_v, acc_sh, sg, ss,
                 wid, s, W):
    zeros16 = jnp.zeros((L,), jnp.float32)

    def zfill(i, carry):
        for j in range(W // L):
            rows_v[0, i, pl.ds(j * L, L)] = zeros16
        return carry

    lax.fori_loop(0, K, zfill, 0)
    r0 = s * ROWS_PT
    for j in range(ROWS_PT // K):
        pltpu.sync_copy(rows_v.at[0], acc_sh.at[pl.ds(r0 + j * K, K)])
    plsc.subcore_barrier()

    for j in range(G):
        pltpu.async_copy(tab_hbm.at[sidx_v.at[j]], rows_v.at[j], sg[j])

    def outer(t, carry):
        for i in range(NB):
            j = t * NB + i
            pltpu.make_async_copy(
                tab_hbm.at[sidx_v.at[j]], rows_v.at[i], sg[i]).wait()
            pltpu.async_copy(rows_v.at[i], acc_sh.at[didx_v.at[j]], ss[i], add=True)
            jj = j + G
            bg = (i + G) % NB

            def start_gather():
                pltpu.make_async_copy(
                    rows_v.at[bg], acc_sh.at[didx_v.at[j]], ss[bg]).wait()
                pltpu.async_copy(tab_hbm.at[sidx_v.at[jj]], rows_v.at[bg], sg[bg])

            if i < NB - G:
                @pl.when(t > 0)
                def _():
                    start_gather()

                @pl.when(jnp.logical_and(t == 0, jj < NCHUNK))
                def _():
                    pltpu.async_copy(tab_hbm.at[sidx_v.at[jj]], rows_v.at[bg], sg[bg])
            else:
                @pl.when(jj < NCHUNK)
                def _():
                    start_gather()
        return carry

    lax.fori_loop(0, NT, outer, 0)
    for i in range(NB):
        pltpu.make_async_copy(rows_v.at[i], acc_sh.at[didx_v.at[i]], ss[i]).wait()

    @pl.when(wid < NTAIL)
    def _():
        pltpu.async_copy(tab_hbm.at[tsx_v], rows_v.at[0], sg[0]).wait()
        pltpu.sync_copy(rows_v.at[0], acc_sh.at[tdx_v], add=True)

    plsc.subcore_barrier()


def _stage_indices(src_hbm, dst_hbm, sidx_v, didx_v, tsx_v, tdx_v, wid, sg):
    crow = wid * NCHUNK
    pltpu.async_copy(src_hbm.at[pl.ds(crow, NCHUNK)], sidx_v, sg[0])
    pltpu.async_copy(dst_hbm.at[pl.ds(crow, NCHUNK)], didx_v, sg[1])

    @pl.when(wid < NTAIL)
    def _():
        trow = NW * NCHUNK + wid
        pltpu.sync_copy(src_hbm.at[trow], tsx_v)
        pltpu.sync_copy(dst_hbm.at[trow], tdx_v)

    pltpu.make_async_copy(src_hbm.at[pl.ds(crow, NCHUNK)], sidx_v, sg[0]).wait()
    pltpu.make_async_copy(dst_hbm.at[pl.ds(crow, NCHUNK)], didx_v, sg[1]).wait()


def _sc_segsum(table, src2d, dst2d, W):
    def body(tab_hbm, src_hbm, dst_hbm, out_hbm, sidx_v, didx_v, tsx_v, tdx_v,
             rows_v, acc_sh, *sems):
        sg = sems[:NB]
        ss = sems[NB:]
        c = lax.axis_index("c")
        s = lax.axis_index("s")
        wid = c * NS + s
        _stage_indices(src_hbm, dst_hbm, sidx_v, didx_v, tsx_v, tdx_v, wid, sg)
        _segsum_core(tab_hbm, sidx_v, didx_v, tsx_v, tdx_v, rows_v, acc_sh,
                     sg, ss, wid, s, W)
        r0 = s * ROWS_PT
        pltpu.sync_copy(acc_sh.at[pl.ds(r0, ROWS_PT)], out_hbm.at[c, pl.ds(r0, ROWS_PT)])

    return pl.kernel(
        body,
        out_type=jax.ShapeDtypeStruct((NC, NP, W), jnp.float32),
        mesh=_sc_mesh(),
        compiler_params=_SC_PARAMS,
        scratch_types=[
            pltpu.VMEM((NCHUNK, K), jnp.int32),
            pltpu.VMEM((NCHUNK, K), jnp.int32),
            pltpu.VMEM((K,), jnp.int32),
            pltpu.VMEM((K,), jnp.int32),
            pltpu.VMEM((NB, K, W), jnp.float32),
            pltpu.VMEM_SHARED((NP, W), jnp.float32),
        ] + [pltpu.SemaphoreType.DMA] * (2 * NB),
    )(table, src2d, dst2d)


def _sc_segsum_sample(table, src2d, dst2d, ab, samp):
    W = LATENT
    SGT = SP // NS

    def body(tab_hbm, src_hbm, dst_hbm, ab_hbm, samp_hbm, g_hbm, gab_hbm,
             sidx_v, didx_v, tsx_v, tdx_v, rows_v, smp_v, gr_v, gab_v, acc_sh,
             *sems):
        sg = sems[:NB]
        ss = sems[NB:]
        c = lax.axis_index("c")
        s = lax.axis_index("s")
        wid = c * NS + s
        pltpu.sync_copy(samp_hbm, smp_v)
        _stage_indices(src_hbm, dst_hbm, sidx_v, didx_v, tsx_v, tdx_v, wid, sg)
        _segsum_core(tab_hbm, sidx_v, didx_v, tsx_v, tdx_v, rows_v, acc_sh,
                     sg, ss, wid, s, W)
        srow = s * SGT
        pltpu.async_copy(acc_sh.at[smp_v.at[pl.ds(srow, SGT)]], gr_v, sg[0])
        abase = wid * SPT
        pltpu.async_copy(ab_hbm.at[smp_v.at[pl.ds(abase, SPT)]], gab_v, sg[1])
        pltpu.make_async_copy(acc_sh.at[smp_v.at[pl.ds(srow, SGT)]], gr_v, sg[0]).wait()
        pltpu.make_async_copy(ab_hbm.at[smp_v.at[pl.ds(abase, SPT)]], gab_v, sg[1]).wait()
        pltpu.sync_copy(gr_v, g_hbm.at[c, pl.ds(srow, SGT)])
        pltpu.sync_copy(gab_v, gab_hbm.at[pl.ds(abase, SPT)])

    return pl.kernel(
        body,
        out_type=[
            jax.ShapeDtypeStruct((NC, SP, LATENT), jnp.float32),
            jax.ShapeDtypeStruct((SP, AB), jnp.float32),
        ],
        mesh=_sc_mesh(),
        compiler_params=_SC_PARAMS,
        scratch_types=[
            pltpu.VMEM((NCHUNK, K), jnp.int32),
            pltpu.VMEM((NCHUNK, K), jnp.int32),
            pltpu.VMEM((K,), jnp.int32),
            pltpu.VMEM((K,), jnp.int32),
            pltpu.VMEM((NB, K, W), jnp.float32),
            pltpu.VMEM((SP,), jnp.int32),
            pltpu.VMEM((SP // NS, LATENT), jnp.float32),
            pltpu.VMEM((SPT, AB), jnp.float32),
            pltpu.VMEM_SHARED((NP, W), jnp.float32),
        ] + [pltpu.SemaphoreType.DMA] * (2 * NB),
    )(table, src2d, dst2d, ab, samp)


def _tc_xw(x, W1):
    BM = 2000

    def body(x_ref, w_ref, o_ref):
        o_ref[...] = jnp.dot(x_ref[...], w_ref[...],
                             preferred_element_type=jnp.float32)

    return pl.pallas_call(
        body,
        grid=(N // BM,),
        in_specs=[
            pl.BlockSpec((BM, D_IN), lambda i: (i, 0)),
            pl.BlockSpec((D_IN, HIDDEN), lambda i: (0, 0)),
        ],
        out_specs=pl.BlockSpec((BM, HIDDEN), lambda i: (i, 0)),
        out_shape=jax.ShapeDtypeStruct((N, HIDDEN), jnp.float32),
    )(x, W1)


def _tc_scale(xw, degt):
    BM = 2000

    def body(xw_ref, d_ref, h_ref, ab_ref):
        dout = d_ref[:, 0:1] + d_ref[:, 1:2]
        din = d_ref[:, 2:3] + d_ref[:, 3:4]
        a = lax.rsqrt(jnp.maximum(dout, 1.0))
        b = lax.rsqrt(jnp.maximum(din, 1.0))
        h_ref[...] = xw_ref[...] * a
        ab_ref[...] = jnp.concatenate(
            [a, b, jnp.zeros((BM, AB - 2), jnp.float32)], axis=1)

    return pl.pallas_call(
        body,
        grid=(N // BM,),
        in_specs=[
            pl.BlockSpec((BM, HIDDEN), lambda i: (i, 0)),
            pl.BlockSpec((BM, 4), lambda i: (i, 0)),
        ],
        out_specs=[
            pl.BlockSpec((BM, HIDDEN), lambda i: (i, 0)),
            pl.BlockSpec((BM, AB), lambda i: (i, 0)),
        ],
        out_shape=[
            jax.ShapeDtypeStruct((N, HIDDEN), jnp.float32),
            jax.ShapeDtypeStruct((N, AB), jnp.float32),
        ],
    )(xw, degt)


def _tc_mid(s1p, ab, W2):
    BM = 2000

    def body(p0_ref, p1_ref, ab_ref, w_ref, o_ref):
        b = ab_ref[:, 1:2]
        a = ab_ref[:, 0:1]
        hidden = jnp.maximum((p0_ref[0] + p1_ref[0]) * b, 0.0)
        o_ref[...] = jnp.dot(hidden, w_ref[...], preferred_element_type=jnp.float32) * a

    return pl.pallas_call(
        body,
        grid=(N // BM,),
        in_specs=[
            pl.BlockSpec((1, BM, HIDDEN), lambda i: (0, i, 0)),
            pl.BlockSpec((1, BM, HIDDEN), lambda i: (1, i, 0)),
            pl.BlockSpec((BM, AB), lambda i: (i, 0)),
            pl.BlockSpec((HIDDEN, LATENT), lambda i: (0, 0)),
        ],
        out_specs=pl.BlockSpec((BM, LATENT), lambda i: (i, 0)),
        out_shape=jax.ShapeDtypeStruct((N, LATENT), jnp.float32),
    )(s1p, s1p, ab, W2)


def _tc_decoder(g, gab):
    R = 200

    def body(g_ref, gab_ref, o_ref):
        i = pl.program_id(0)
        gb = gab_ref[:, 1:2]
        z = (g_ref[0] + g_ref[1]) * gb
        zr = (g_ref[0, pl.ds(i * R, R), :] + g_ref[1, pl.ds(i * R, R), :]) \
            * gab_ref[pl.ds(i * R, R), 1:2]
        gram = lax.dot_general(zr, z, (((1,), (1,)), ((), ())),
                               preferred_element_type=jnp.float32)
        zz = z * z
        sqc = lax.dot_general(jnp.ones((1, LATENT), jnp.float32), zz,
                              (((1,), (1,)), ((), ())),
                              preferred_element_type=jnp.float32)
        sqr = jnp.sum(zr * zr, axis=1, keepdims=True)
        d2 = jnp.maximum(sqr + sqc - 2.0 * gram, 0.0)
        o_ref[0] = gram
        o_ref[1] = jnp.sqrt(d2 + 1e-12)

    return pl.pallas_call(
        body,
        grid=(S // R,),
        in_specs=[
            pl.BlockSpec((NC, S, LATENT), lambda i: (0, 0, 0)),
            pl.BlockSpec((S, AB), lambda i: (0, 0)),
        ],
        out_specs=pl.BlockSpec((2, R, S), lambda i: (0, i, 0)),
        out_shape=jax.ShapeDtypeStruct((2, S, S), jnp.float32),
    )(g, gab)


@jax.jit
def kernel(x, edge_index, sampled_nodes, W1, W2):
    src2d = edge_index[0].reshape(R2D, K)
    dst2d = edge_index[1].reshape(R2D, K)
    xw = _tc_xw(x, W1)
    deg_out, deg_in = _sc_degrees(src2d, dst2d)
    degt = jnp.concatenate([deg_out[:, :N], deg_in[:, :N]], axis=0).T
    h1p, ab = _tc_scale(xw, degt)
    s1p = _sc_segsum(h1p, src2d, dst2d, HIDDEN)
    h2p = _tc_mid(s1p, ab, W2)
    samp = jnp.concatenate([sampled_nodes, jnp.zeros((SP - S,), jnp.int32)])
    g, gab = _sc_segsum_sample(h2p, src2d, dst2d, ab, samp)
    out = _tc_decoder(g[:, :S], gab[:S])
    return out.reshape(2, S * S)


# speedup vs baseline: 44.4409x; 1.0282x over previous
import jax
import jax.numpy as jnp
from jax import lax
from jax.experimental import pallas as pl
from jax.experimental.pallas import tpu as pltpu
from jax.experimental.pallas import tpu_sc as plsc

N = 10000
E = 320000
D_IN = 128
HIDDEN = 32
LATENT = 16
S = 1000

NC = 2
NS = 16
NW = NC * NS
L = 16

NP = 10240
ROWS_PT = NP // NS
K = 256
R2D = E // K
NCHUNK = 39
NTAIL = R2D - NW * NCHUNK
NB = 3
NT = NCHUNK // NB
MC = 1
NM = NCHUNK // MC
NBM = 3
GM = 2
NTM = NM // NBM
SP = 1024
SPT = SP // NW
AB = 8


def _sc_mesh():
    return plsc.VectorSubcoreMesh(
        core_axis_name="c", subcore_axis_name="s", num_cores=NC, num_subcores=NS
    )


_SC_PARAMS = pltpu.CompilerParams(use_tc_tiling_on_sc=False)


def _sc_degrees(src2d, dst2d):
    def body(src_hbm, dst_hbm, do_hbm, di_hbm, sidx_v, didx_v, tsx_v, tdx_v,
             ones_v, zrow_v, do_sh, di_sh, *sems):
        sa = sems[:NB]
        sb = sems[NB:]
        c = lax.axis_index("c")
        s = lax.axis_index("s")
        wid = c * NS + s
        zeros16 = jnp.zeros((L,), jnp.float32)
        ones16 = jnp.ones((L,), jnp.float32)
        for i in range(K // L):
            ones_v[pl.ds(i * L, L)] = ones16

        def zfill(i, carry):
            zrow_v[pl.ds(i * L, L)] = zeros16
            return carry

        lax.fori_loop(0, ROWS_PT // L, zfill, 0)
        r0 = s * ROWS_PT
        pltpu.sync_copy(zrow_v, do_sh.at[pl.ds(r0, ROWS_PT)])
        pltpu.sync_copy(zrow_v, di_sh.at[pl.ds(r0, ROWS_PT)])
        crow = wid * NCHUNK
        pltpu.sync_copy(src_hbm.at[pl.ds(crow, NCHUNK)], sidx_v)
        pltpu.sync_copy(dst_hbm.at[pl.ds(crow, NCHUNK)], didx_v)

        @pl.when(wid < NTAIL)
        def _():
            trow = NW * NCHUNK + wid
            pltpu.sync_copy(src_hbm.at[trow], tsx_v)
            pltpu.sync_copy(dst_hbm.at[trow], tdx_v)

        plsc.subcore_barrier()

        def outer(t, carry):
            for i in range(NB):
                j = t * NB + i

                @pl.when(t > 0)
                def _():
                    pltpu.make_async_copy(ones_v, do_sh.at[sidx_v.at[j]], sa[i]).wait()
                    pltpu.make_async_copy(ones_v, di_sh.at[didx_v.at[j]], sb[i]).wait()

                pltpu.async_copy(ones_v, do_sh.at[sidx_v.at[j]], sa[i], add=True)
                pltpu.async_copy(ones_v, di_sh.at[didx_v.at[j]], sb[i], add=True)
            return carry

        lax.fori_loop(0, NT, outer, 0)
        for i in range(NB):
            pltpu.make_async_copy(ones_v, do_sh.at[sidx_v.at[i]], sa[i]).wait()
            pltpu.make_async_copy(ones_v, di_sh.at[didx_v.at[i]], sb[i]).wait()

        @pl.when(wid < NTAIL)
        def _():
            pltpu.sync_copy(ones_v, do_sh.at[tsx_v], add=True)
            pltpu.sync_copy(ones_v, di_sh.at[tdx_v], add=True)

        plsc.subcore_barrier()
        pltpu.sync_copy(do_sh.at[pl.ds(r0, ROWS_PT)], do_hbm.at[c, pl.ds(r0, ROWS_PT)])
        pltpu.sync_copy(di_sh.at[pl.ds(r0, ROWS_PT)], di_hbm.at[c, pl.ds(r0, ROWS_PT)])

    return pl.kernel(
        body,
        out_type=[
            jax.ShapeDtypeStruct((NC, NP), jnp.float32),
            jax.ShapeDtypeStruct((NC, NP), jnp.float32),
        ],
        mesh=_sc_mesh(),
        compiler_params=_SC_PARAMS,
        scratch_types=[
            pltpu.VMEM((NCHUNK, K), jnp.int32),
            pltpu.VMEM((NCHUNK, K), jnp.int32),
            pltpu.VMEM((K,), jnp.int32),
            pltpu.VMEM((K,), jnp.int32),
            pltpu.VMEM((K,), jnp.float32),
            pltpu.VMEM((ROWS_PT,), jnp.float32),
            pltpu.VMEM_SHARED((NP,), jnp.float32),
            pltpu.VMEM_SHARED((NP,), jnp.float32),
        ] + [pltpu.SemaphoreType.DMA] * (2 * NB),
    )(src2d, dst2d)


def _segsum_core(tab_hbm, sidx_v, didx_v, tsx_v, tdx_v, rows_v, acc_sh, sg, ss,
                 wid, s, W):
    zeros16 = jnp.zeros((L,), jnp.float32)

    def zfill(i, carry):
        for j in range(W // L):
            rows_v[0, i, pl.ds(j * L, L)] = zeros16
        return carry

    lax.fori_loop(0, K, zfill, 0)
    r0 = s * ROWS_PT
    for j in range(ROWS_PT // K):
        pltpu.sync_copy(rows_v.at[0], acc_sh.at[pl.ds(r0 + j * K, K)])
    rem = ROWS_PT % K
    if rem:
        pltpu.sync_copy(rows_v.at[0].at[pl.ds(0, rem)],
                        acc_sh.at[pl.ds(r0 + ROWS_PT - rem, rem)])
    plsc.subcore_barrier()

    def sidx_at(j):
        return sidx_v.at[j]

    def didx_at(j):
        return didx_v.at[j]

    for j in range(GM):
        pltpu.async_copy(tab_hbm.at[sidx_at(j)], rows_v.at[j], sg[j])

    def outer(t, carry):
        for i in range(NBM):
            j = t * NBM + i
            pltpu.make_async_copy(
                tab_hbm.at[sidx_at(j)], rows_v.at[i], sg[i]).wait()
            pltpu.async_copy(rows_v.at[i], acc_sh.at[didx_at(j)], ss[i], add=True)
            jj = j + GM
            bg = (i + GM) % NBM

            def start_gather():
                pltpu.make_async_copy(
                    rows_v.at[bg], acc_sh.at[didx_at(j)], ss[bg]).wait()
                pltpu.async_copy(tab_hbm.at[sidx_at(jj)], rows_v.at[bg], sg[bg])

            if i < NBM - GM:
                @pl.when(t > 0)
                def _():
                    start_gather()

                @pl.when(jnp.logical_and(t == 0, jj < NM))
                def _():
                    pltpu.async_copy(tab_hbm.at[sidx_at(jj)], rows_v.at[bg], sg[bg])
            else:
                @pl.when(jj < NM)
                def _():
                    start_gather()
        return carry

    lax.fori_loop(0, NTM, outer, 0)
    for i in range(NBM):
        pltpu.make_async_copy(rows_v.at[i], acc_sh.at[didx_at(i)], ss[i]).wait()

    @pl.when(wid < NTAIL)
    def _():
        pltpu.async_copy(tab_hbm.at[tsx_v], rows_v.at[0], sg[0]).wait()
        pltpu.sync_copy(rows_v.at[0], acc_sh.at[tdx_v], add=True)

    plsc.subcore_barrier()


def _stage_indices(src_hbm, dst_hbm, sidx_v, didx_v, tsx_v, tdx_v, wid, sg):
    crow = wid * NCHUNK
    pltpu.async_copy(src_hbm.at[pl.ds(crow, NCHUNK)], sidx_v, sg[0])
    pltpu.async_copy(dst_hbm.at[pl.ds(crow, NCHUNK)], didx_v, sg[1])

    @pl.when(wid < NTAIL)
    def _():
        trow = NW * NCHUNK + wid
        pltpu.sync_copy(src_hbm.at[trow], tsx_v)
        pltpu.sync_copy(dst_hbm.at[trow], tdx_v)

    pltpu.make_async_copy(src_hbm.at[pl.ds(crow, NCHUNK)], sidx_v, sg[0]).wait()
    pltpu.make_async_copy(dst_hbm.at[pl.ds(crow, NCHUNK)], didx_v, sg[1]).wait()


def _sc_segsum(table, src2d, dst2d, W):
    def body(tab_hbm, src_hbm, dst_hbm, out_hbm, sidx_v, didx_v, tsx_v, tdx_v,
             rows_v, acc_sh, *sems):
        sg = sems[:NBM]
        ss = sems[NBM:]
        c = lax.axis_index("c")
        s = lax.axis_index("s")
        wid = c * NS + s
        _stage_indices(src_hbm, dst_hbm, sidx_v, didx_v, tsx_v, tdx_v, wid, sg)
        _segsum_core(tab_hbm, sidx_v, didx_v, tsx_v, tdx_v, rows_v, acc_sh,
                     sg, ss, wid, s, W)
        r0 = s * ROWS_PT
        pltpu.sync_copy(acc_sh.at[pl.ds(r0, ROWS_PT)], out_hbm.at[c, pl.ds(r0, ROWS_PT)])

    return pl.kernel(
        body,
        out_type=jax.ShapeDtypeStruct((NC, NP, W), jnp.float32),
        mesh=_sc_mesh(),
        compiler_params=_SC_PARAMS,
        scratch_types=[
            pltpu.VMEM((NCHUNK, K), jnp.int32),
            pltpu.VMEM((NCHUNK, K), jnp.int32),
            pltpu.VMEM((K,), jnp.int32),
            pltpu.VMEM((K,), jnp.int32),
            pltpu.VMEM((NBM, K, W), jnp.float32),
            pltpu.VMEM_SHARED((NP, W), jnp.float32),
        ] + [pltpu.SemaphoreType.DMA] * (2 * NBM),
    )(table, src2d, dst2d)


def _sc_segsum_sample(table, src2d, dst2d, ab, samp):
    W = LATENT
    SGT = SP // NS

    def body(tab_hbm, src_hbm, dst_hbm, ab_hbm, samp_hbm, g_hbm, gab_hbm,
             sidx_v, didx_v, tsx_v, tdx_v, rows_v, smp_v, gr_v, gab_v, acc_sh,
             *sems):
        sg = sems[:NBM]
        ss = sems[NBM:]
        c = lax.axis_index("c")
        s = lax.axis_index("s")
        wid = c * NS + s
        pltpu.sync_copy(samp_hbm, smp_v)
        _stage_indices(src_hbm, dst_hbm, sidx_v, didx_v, tsx_v, tdx_v, wid, sg)
        _segsum_core(tab_hbm, sidx_v, didx_v, tsx_v, tdx_v, rows_v, acc_sh,
                     sg, ss, wid, s, W)
        srow = s * SGT
        pltpu.async_copy(acc_sh.at[smp_v.at[pl.ds(srow, SGT)]], gr_v, sg[0])
        abase = wid * SPT
        pltpu.async_copy(ab_hbm.at[smp_v.at[pl.ds(abase, SPT)]], gab_v, sg[1])
        pltpu.make_async_copy(acc_sh.at[smp_v.at[pl.ds(srow, SGT)]], gr_v, sg[0]).wait()
        pltpu.make_async_copy(ab_hbm.at[smp_v.at[pl.ds(abase, SPT)]], gab_v, sg[1]).wait()
        pltpu.sync_copy(gr_v, g_hbm.at[c, pl.ds(srow, SGT)])
        pltpu.sync_copy(gab_v, gab_hbm.at[pl.ds(abase, SPT)])

    return pl.kernel(
        body,
        out_type=[
            jax.ShapeDtypeStruct((NC, SP, LATENT), jnp.float32),
            jax.ShapeDtypeStruct((SP, AB), jnp.float32),
        ],
        mesh=_sc_mesh(),
        compiler_params=_SC_PARAMS,
        scratch_types=[
            pltpu.VMEM((NCHUNK, K), jnp.int32),
            pltpu.VMEM((NCHUNK, K), jnp.int32),
            pltpu.VMEM((K,), jnp.int32),
            pltpu.VMEM((K,), jnp.int32),
            pltpu.VMEM((NBM, K, W), jnp.float32),
            pltpu.VMEM((SP,), jnp.int32),
            pltpu.VMEM((SP // NS, LATENT), jnp.float32),
            pltpu.VMEM((SPT, AB), jnp.float32),
            pltpu.VMEM_SHARED((NP, W), jnp.float32),
        ] + [pltpu.SemaphoreType.DMA] * (2 * NBM),
    )(table, src2d, dst2d, ab, samp)


def _tc_xw(x, W1):
    BM = 2000

    def body(x_ref, w_ref, o_ref):
        o_ref[...] = jnp.dot(x_ref[...], w_ref[...],
                             preferred_element_type=jnp.float32)

    return pl.pallas_call(
        body,
        grid=(N // BM,),
        in_specs=[
            pl.BlockSpec((BM, D_IN), lambda i: (i, 0)),
            pl.BlockSpec((D_IN, HIDDEN), lambda i: (0, 0)),
        ],
        out_specs=pl.BlockSpec((BM, HIDDEN), lambda i: (i, 0)),
        out_shape=jax.ShapeDtypeStruct((N, HIDDEN), jnp.float32),
    )(x, W1)


def _tc_scale(xw, degt):
    BM = 2000

    def body(xw_ref, d_ref, h_ref, ab_ref):
        dout = d_ref[:, 0:1] + d_ref[:, 1:2]
        din = d_ref[:, 2:3] + d_ref[:, 3:4]
        a = lax.rsqrt(jnp.maximum(dout, 1.0))
        b = lax.rsqrt(jnp.maximum(din, 1.0))
        h_ref[...] = xw_ref[...] * a
        ab_ref[...] = jnp.concatenate(
            [a, b, jnp.zeros((BM, AB - 2), jnp.float32)], axis=1)

    return pl.pallas_call(
        body,
        grid=(N // BM,),
        in_specs=[
            pl.BlockSpec((BM, HIDDEN), lambda i: (i, 0)),
            pl.BlockSpec((BM, 4), lambda i: (i, 0)),
        ],
        out_specs=[
            pl.BlockSpec((BM, HIDDEN), lambda i: (i, 0)),
            pl.BlockSpec((BM, AB), lambda i: (i, 0)),
        ],
        out_shape=[
            jax.ShapeDtypeStruct((N, HIDDEN), jnp.float32),
            jax.ShapeDtypeStruct((N, AB), jnp.float32),
        ],
    )(xw, degt)


def _tc_mid(s1p, ab, W2):
    BM = 2000

    def body(p0_ref, p1_ref, ab_ref, w_ref, o_ref):
        b = ab_ref[:, 1:2]
        a = ab_ref[:, 0:1]
        hidden = jnp.maximum((p0_ref[0] + p1_ref[0]) * b, 0.0)
        o_ref[...] = jnp.dot(hidden, w_ref[...], preferred_element_type=jnp.float32) * a

    return pl.pallas_call(
        body,
        grid=(N // BM,),
        in_specs=[
            pl.BlockSpec((1, BM, HIDDEN), lambda i: (0, i, 0)),
            pl.BlockSpec((1, BM, HIDDEN), lambda i: (1, i, 0)),
            pl.BlockSpec((BM, AB), lambda i: (i, 0)),
            pl.BlockSpec((HIDDEN, LATENT), lambda i: (0, 0)),
        ],
        out_specs=pl.BlockSpec((BM, LATENT), lambda i: (i, 0)),
        out_shape=jax.ShapeDtypeStruct((N, LATENT), jnp.float32),
    )(s1p, s1p, ab, W2)


def _tc_decoder(g, gab):
    R = 200

    def body(g_ref, gab_ref, o_ref):
        i = pl.program_id(0)
        gb = gab_ref[:, 1:2]
        z = (g_ref[0] + g_ref[1]) * gb
        zr = (g_ref[0, pl.ds(i * R, R), :] + g_ref[1, pl.ds(i * R, R), :]) \
            * gab_ref[pl.ds(i * R, R), 1:2]
        gram = lax.dot_general(zr, z, (((1,), (1,)), ((), ())),
                               preferred_element_type=jnp.float32)
        zz = z * z
        sqc = lax.dot_general(jnp.ones((1, LATENT), jnp.float32), zz,
                              (((1,), (1,)), ((), ())),
                              preferred_element_type=jnp.float32)
        sqr = jnp.sum(zr * zr, axis=1, keepdims=True)
        d2 = jnp.maximum(sqr + sqc - 2.0 * gram, 0.0)
        o_ref[0] = gram
        o_ref[1] = jnp.sqrt(d2 + 1e-12)

    return pl.pallas_call(
        body,
        grid=(S // R,),
        in_specs=[
            pl.BlockSpec((NC, S, LATENT), lambda i: (0, 0, 0)),
            pl.BlockSpec((S, AB), lambda i: (0, 0)),
        ],
        out_specs=pl.BlockSpec((2, R, S), lambda i: (0, i, 0)),
        out_shape=jax.ShapeDtypeStruct((2, S, S), jnp.float32),
    )(g, gab)


@jax.jit
def kernel(x, edge_index, sampled_nodes, W1, W2):
    src2d = edge_index[0].reshape(R2D, K)
    dst2d = edge_index[1].reshape(R2D, K)
    xw = _tc_xw(x, W1)
    deg_out, deg_in = _sc_degrees(src2d, dst2d)
    degt = jnp.concatenate([deg_out[:, :N], deg_in[:, :N]], axis=0).T
    h1p, ab = _tc_scale(xw, degt)
    s1p = _sc_segsum(h1p, src2d, dst2d, HIDDEN)
    h2p = _tc_mid(s1p, ab, W2)
    samp = jnp.concatenate([sampled_nodes, jnp.zeros((SP - S,), jnp.int32)])
    g, gab = _sc_segsum_sample(h2p, src2d, dst2d, ab, samp)
    out = _tc_decoder(g[:, :S], gab[:S])
    return out.reshape(2, S * S)


# speedup vs baseline: 45.4633x; 1.0230x over previous
import jax
import jax.numpy as jnp
from jax import lax
from jax.experimental import pallas as pl
from jax.experimental.pallas import tpu as pltpu
from jax.experimental.pallas import tpu_sc as plsc

N = 10000
E = 320000
D_IN = 128
HIDDEN = 32
LATENT = 16
S = 1000

NC = 2
NS = 16
NW = NC * NS
L = 16

NP = 10240
ROWS_PT = NP // NS
K = 256
R2D = E // K
NCHUNK = 39
NTAIL = R2D - NW * NCHUNK
NB = 3
NT = NCHUNK // NB
MC = 1
NM = NCHUNK // MC
NBM = 3
GM = 2
NTM = NM // NBM
SP = 1024
SPT = SP // NW
AB = 8


def _sc_mesh():
    return plsc.VectorSubcoreMesh(
        core_axis_name="c", subcore_axis_name="s", num_cores=NC, num_subcores=NS
    )


_SC_PARAMS = pltpu.CompilerParams(use_tc_tiling_on_sc=False)


def _sc_degrees(src2d, dst2d):
    def body(src_hbm, dst_hbm, do_hbm, di_hbm, sidx_v, didx_v, tsx_v, tdx_v,
             ones_v, zrow_v, do_sh, di_sh, *sems):
        sa = sems[:NB]
        sb = sems[NB:]
        c = lax.axis_index("c")
        s = lax.axis_index("s")
        wid = c * NS + s
        zeros16 = jnp.zeros((L,), jnp.float32)
        ones16 = jnp.ones((L,), jnp.float32)
        for i in range(K // L):
            ones_v[pl.ds(i * L, L)] = ones16

        def zfill(i, carry):
            zrow_v[pl.ds(i * L, L)] = zeros16
            return carry

        lax.fori_loop(0, ROWS_PT // L, zfill, 0)
        r0 = s * ROWS_PT
        pltpu.sync_copy(zrow_v, do_sh.at[pl.ds(r0, ROWS_PT)])
        pltpu.sync_copy(zrow_v, di_sh.at[pl.ds(r0, ROWS_PT)])
        crow = wid * NCHUNK
        pltpu.sync_copy(src_hbm.at[pl.ds(crow, NCHUNK)], sidx_v)
        pltpu.sync_copy(dst_hbm.at[pl.ds(crow, NCHUNK)], didx_v)

        @pl.when(wid < NTAIL)
        def _():
            trow = NW * NCHUNK + wid
            pltpu.sync_copy(src_hbm.at[trow], tsx_v)
            pltpu.sync_copy(dst_hbm.at[trow], tdx_v)

        plsc.subcore_barrier()

        def outer(t, carry):
            for i in range(NB):
                j = t * NB + i

                @pl.when(t > 0)
                def _():
                    pltpu.make_async_copy(ones_v, do_sh.at[sidx_v.at[j]], sa[i]).wait()
                    pltpu.make_async_copy(ones_v, di_sh.at[didx_v.at[j]], sb[i]).wait()

                pltpu.async_copy(ones_v, do_sh.at[sidx_v.at[j]], sa[i], add=True)
                pltpu.async_copy(ones_v, di_sh.at[didx_v.at[j]], sb[i], add=True)
            return carry

        lax.fori_loop(0, NT, outer, 0)
        for i in range(NB):
            pltpu.make_async_copy(ones_v, do_sh.at[sidx_v.at[i]], sa[i]).wait()
            pltpu.make_async_copy(ones_v, di_sh.at[didx_v.at[i]], sb[i]).wait()

        @pl.when(wid < NTAIL)
        def _():
            pltpu.sync_copy(ones_v, do_sh.at[tsx_v], add=True)
            pltpu.sync_copy(ones_v, di_sh.at[tdx_v], add=True)

        plsc.subcore_barrier()
        pltpu.sync_copy(do_sh.at[pl.ds(r0, ROWS_PT)], do_hbm.at[c, pl.ds(r0, ROWS_PT)])
        pltpu.sync_copy(di_sh.at[pl.ds(r0, ROWS_PT)], di_hbm.at[c, pl.ds(r0, ROWS_PT)])

    return pl.kernel(
        body,
        out_type=[
            jax.ShapeDtypeStruct((NC, NP), jnp.float32),
            jax.ShapeDtypeStruct((NC, NP), jnp.float32),
        ],
        mesh=_sc_mesh(),
        compiler_params=_SC_PARAMS,
        scratch_types=[
            pltpu.VMEM((NCHUNK, K), jnp.int32),
            pltpu.VMEM((NCHUNK, K), jnp.int32),
            pltpu.VMEM((K,), jnp.int32),
            pltpu.VMEM((K,), jnp.int32),
            pltpu.VMEM((K,), jnp.float32),
            pltpu.VMEM((ROWS_PT,), jnp.float32),
            pltpu.VMEM_SHARED((NP,), jnp.float32),
            pltpu.VMEM_SHARED((NP,), jnp.float32),
        ] + [pltpu.SemaphoreType.DMA] * (2 * NB),
    )(src2d, dst2d)


def _segsum_core(tab_hbm, sidx_v, didx_v, tsx_v, tdx_v, rows_v, acc_sh, sg, ss,
                 wid, s, W):
    zeros16 = jnp.zeros((L,), jnp.float32)

    def zfill(i, carry):
        for j in range(W // L):
            rows_v[0, i, pl.ds(j * L, L)] = zeros16
        return carry

    lax.fori_loop(0, K, zfill, 0)
    r0 = s * ROWS_PT
    for j in range(ROWS_PT // K):
        pltpu.sync_copy(rows_v.at[0], acc_sh.at[pl.ds(r0 + j * K, K)])
    rem = ROWS_PT % K
    if rem:
        pltpu.sync_copy(rows_v.at[0].at[pl.ds(0, rem)],
                        acc_sh.at[pl.ds(r0 + ROWS_PT - rem, rem)])
    plsc.subcore_barrier()

    def sidx_at(j):
        return sidx_v.at[j]

    def didx_at(j):
        return didx_v.at[j]

    for j in range(GM):
        pltpu.async_copy(tab_hbm.at[sidx_at(j)], rows_v.at[j], sg[j])

    def outer(t, carry):
        for i in range(NBM):
            j = t * NBM + i
            pltpu.make_async_copy(
                tab_hbm.at[sidx_at(j)], rows_v.at[i], sg[i]).wait()
            pltpu.async_copy(rows_v.at[i], acc_sh.at[didx_at(j)], ss[i], add=True)
            jj = j + GM
            bg = (i + GM) % NBM

            def start_gather():
                pltpu.make_async_copy(
                    rows_v.at[bg], acc_sh.at[didx_at(j)], ss[bg]).wait()
                pltpu.async_copy(tab_hbm.at[sidx_at(jj)], rows_v.at[bg], sg[bg])

            if i < NBM - GM:
                @pl.when(t > 0)
                def _():
                    start_gather()

                @pl.when(jnp.logical_and(t == 0, jj < NM))
                def _():
                    pltpu.async_copy(tab_hbm.at[sidx_at(jj)], rows_v.at[bg], sg[bg])
            else:
                @pl.when(jj < NM)
                def _():
                    start_gather()
        return carry

    lax.fori_loop(0, NTM, outer, 0)
    for i in range(NBM):
        pltpu.make_async_copy(rows_v.at[i], acc_sh.at[didx_at(i)], ss[i]).wait()

    @pl.when(wid < NTAIL)
    def _():
        pltpu.async_copy(tab_hbm.at[tsx_v], rows_v.at[0], sg[0]).wait()
        pltpu.sync_copy(rows_v.at[0], acc_sh.at[tdx_v], add=True)

    plsc.subcore_barrier()


def _stage_indices(src_hbm, dst_hbm, sidx_v, didx_v, tsx_v, tdx_v, wid, sg):
    crow = wid * NCHUNK
    pltpu.async_copy(src_hbm.at[pl.ds(crow, NCHUNK)], sidx_v, sg[0])
    pltpu.async_copy(dst_hbm.at[pl.ds(crow, NCHUNK)], didx_v, sg[1])

    @pl.when(wid < NTAIL)
    def _():
        trow = NW * NCHUNK + wid
        pltpu.sync_copy(src_hbm.at[trow], tsx_v)
        pltpu.sync_copy(dst_hbm.at[trow], tdx_v)

    pltpu.make_async_copy(src_hbm.at[pl.ds(crow, NCHUNK)], sidx_v, sg[0]).wait()
    pltpu.make_async_copy(dst_hbm.at[pl.ds(crow, NCHUNK)], didx_v, sg[1]).wait()


def _sc_segsum(table, src2d, dst2d, W):
    def body(tab_hbm, src_hbm, dst_hbm, out_hbm, sidx_v, didx_v, tsx_v, tdx_v,
             rows_v, acc_sh, *sems):
        sg = sems[:NBM]
        ss = sems[NBM:]
        c = lax.axis_index("c")
        s = lax.axis_index("s")
        wid = c * NS + s
        _stage_indices(src_hbm, dst_hbm, sidx_v, didx_v, tsx_v, tdx_v, wid, sg)
        _segsum_core(tab_hbm, sidx_v, didx_v, tsx_v, tdx_v, rows_v, acc_sh,
                     sg, ss, wid, s, W)
        r0 = s * ROWS_PT
        pltpu.sync_copy(acc_sh.at[pl.ds(r0, ROWS_PT)], out_hbm.at[c, pl.ds(r0, ROWS_PT)])

    return pl.kernel(
        body,
        out_type=jax.ShapeDtypeStruct((NC, NP, W), jnp.float32),
        mesh=_sc_mesh(),
        compiler_params=_SC_PARAMS,
        scratch_types=[
            pltpu.VMEM((NCHUNK, K), jnp.int32),
            pltpu.VMEM((NCHUNK, K), jnp.int32),
            pltpu.VMEM((K,), jnp.int32),
            pltpu.VMEM((K,), jnp.int32),
            pltpu.VMEM((NBM, K, W), jnp.float32),
            pltpu.VMEM_SHARED((NP, W), jnp.float32),
        ] + [pltpu.SemaphoreType.DMA] * (2 * NBM),
    )(table, src2d, dst2d)


def _sc_segsum_sample(table, src2d, dst2d, ab, samp):
    W = LATENT
    SGT = SP // NS

    def body(tab_hbm, src_hbm, dst_hbm, ab_hbm, samp_hbm, g_hbm, gab_hbm,
             sidx_v, didx_v, tsx_v, tdx_v, rows_v, smp_v, gr_v, gab_v, acc_sh,
             *sems):
        sg = sems[:NBM]
        ss = sems[NBM:]
        c = lax.axis_index("c")
        s = lax.axis_index("s")
        wid = c * NS + s
        pltpu.sync_copy(samp_hbm, smp_v)
        _stage_indices(src_hbm, dst_hbm, sidx_v, didx_v, tsx_v, tdx_v, wid, sg)
        _segsum_core(tab_hbm, sidx_v, didx_v, tsx_v, tdx_v, rows_v, acc_sh,
                     sg, ss, wid, s, W)
        srow = s * SGT
        pltpu.async_copy(acc_sh.at[smp_v.at[pl.ds(srow, SGT)]], gr_v, sg[0])
        abase = wid * SPT
        pltpu.async_copy(ab_hbm.at[smp_v.at[pl.ds(abase, SPT)]], gab_v, sg[1])
        pltpu.make_async_copy(acc_sh.at[smp_v.at[pl.ds(srow, SGT)]], gr_v, sg[0]).wait()
        pltpu.make_async_copy(ab_hbm.at[smp_v.at[pl.ds(abase, SPT)]], gab_v, sg[1]).wait()
        pltpu.sync_copy(gr_v, g_hbm.at[c, pl.ds(srow, SGT)])
        pltpu.sync_copy(gab_v, gab_hbm.at[pl.ds(abase, SPT)])

    return pl.kernel(
        body,
        out_type=[
            jax.ShapeDtypeStruct((NC, SP, LATENT), jnp.float32),
            jax.ShapeDtypeStruct((SP, AB), jnp.float32),
        ],
        mesh=_sc_mesh(),
        compiler_params=_SC_PARAMS,
        scratch_types=[
            pltpu.VMEM((NCHUNK, K), jnp.int32),
            pltpu.VMEM((NCHUNK, K), jnp.int32),
            pltpu.VMEM((K,), jnp.int32),
            pltpu.VMEM((K,), jnp.int32),
            pltpu.VMEM((NBM, K, W), jnp.float32),
            pltpu.VMEM((SP,), jnp.int32),
            pltpu.VMEM((SP // NS, LATENT), jnp.float32),
            pltpu.VMEM((SPT, AB), jnp.float32),
            pltpu.VMEM_SHARED((NP, W), jnp.float32),
        ] + [pltpu.SemaphoreType.DMA] * (2 * NBM),
    )(table, src2d, dst2d, ab, samp)


def _tc_xw(x, W1):
    BM = 5000

    def body(x_ref, w_ref, o_ref):
        o_ref[...] = jnp.dot(x_ref[...], w_ref[...],
                             preferred_element_type=jnp.float32)

    return pl.pallas_call(
        body,
        grid=(N // BM,),
        in_specs=[
            pl.BlockSpec((BM, D_IN), lambda i: (i, 0)),
            pl.BlockSpec((D_IN, HIDDEN), lambda i: (0, 0)),
        ],
        out_specs=pl.BlockSpec((BM, HIDDEN), lambda i: (i, 0)),
        out_shape=jax.ShapeDtypeStruct((N, HIDDEN), jnp.float32),
    )(x, W1)


def _tc_scale(xw, degt):
    BM = 5000

    def body(xw_ref, d_ref, h_ref, ab_ref):
        dout = d_ref[:, 0:1] + d_ref[:, 1:2]
        din = d_ref[:, 2:3] + d_ref[:, 3:4]
        a = lax.rsqrt(jnp.maximum(dout, 1.0))
        b = lax.rsqrt(jnp.maximum(din, 1.0))
        h_ref[...] = xw_ref[...] * a
        ab_ref[...] = jnp.concatenate(
            [a, b, jnp.zeros((BM, AB - 2), jnp.float32)], axis=1)

    return pl.pallas_call(
        body,
        grid=(N // BM,),
        in_specs=[
            pl.BlockSpec((BM, HIDDEN), lambda i: (i, 0)),
            pl.BlockSpec((BM, 4), lambda i: (i, 0)),
        ],
        out_specs=[
            pl.BlockSpec((BM, HIDDEN), lambda i: (i, 0)),
            pl.BlockSpec((BM, AB), lambda i: (i, 0)),
        ],
        out_shape=[
            jax.ShapeDtypeStruct((N, HIDDEN), jnp.float32),
            jax.ShapeDtypeStruct((N, AB), jnp.float32),
        ],
    )(xw, degt)


def _tc_mid(s1p, ab, W2):
    BM = 5000

    def body(p0_ref, p1_ref, ab_ref, w_ref, o_ref):
        b = ab_ref[:, 1:2]
        a = ab_ref[:, 0:1]
        hidden = jnp.maximum((p0_ref[0] + p1_ref[0]) * b, 0.0)
        o_ref[...] = jnp.dot(hidden, w_ref[...], preferred_element_type=jnp.float32) * a

    return pl.pallas_call(
        body,
        grid=(N // BM,),
        in_specs=[
            pl.BlockSpec((1, BM, HIDDEN), lambda i: (0, i, 0)),
            pl.BlockSpec((1, BM, HIDDEN), lambda i: (1, i, 0)),
            pl.BlockSpec((BM, AB), lambda i: (i, 0)),
            pl.BlockSpec((HIDDEN, LATENT), lambda i: (0, 0)),
        ],
        out_specs=pl.BlockSpec((BM, LATENT), lambda i: (i, 0)),
        out_shape=jax.ShapeDtypeStruct((N, LATENT), jnp.float32),
    )(s1p, s1p, ab, W2)


def _tc_decoder(g, gab):
    R = 200

    def body(g_ref, gab_ref, o_ref):
        i = pl.program_id(0)
        gb = gab_ref[:, 1:2]
        z = (g_ref[0] + g_ref[1]) * gb
        zr = (g_ref[0, pl.ds(i * R, R), :] + g_ref[1, pl.ds(i * R, R), :]) \
            * gab_ref[pl.ds(i * R, R), 1:2]
        gram = lax.dot_general(zr, z, (((1,), (1,)), ((), ())),
                               preferred_element_type=jnp.float32)
        zz = z * z
        sqc = lax.dot_general(jnp.ones((1, LATENT), jnp.float32), zz,
                              (((1,), (1,)), ((), ())),
                              preferred_element_type=jnp.float32)
        sqr = jnp.sum(zr * zr, axis=1, keepdims=True)
        d2 = jnp.maximum(sqr + sqc - 2.0 * gram, 0.0)
        o_ref[0] = gram
        o_ref[1] = jnp.sqrt(d2 + 1e-12)

    return pl.pallas_call(
        body,
        grid=(S // R,),
        in_specs=[
            pl.BlockSpec((NC, S, LATENT), lambda i: (0, 0, 0)),
            pl.BlockSpec((S, AB), lambda i: (0, 0)),
        ],
        out_specs=pl.BlockSpec((2, R, S), lambda i: (0, i, 0)),
        out_shape=jax.ShapeDtypeStruct((2, S, S), jnp.float32),
    )(g, gab)


@jax.jit
def kernel(x, edge_index, sampled_nodes, W1, W2):
    src2d = edge_index[0].reshape(R2D, K)
    dst2d = edge_index[1].reshape(R2D, K)
    xw = _tc_xw(x, W1)
    deg_out, deg_in = _sc_degrees(src2d, dst2d)
    degt = jnp.concatenate([deg_out[:, :N], deg_in[:, :N]], axis=0).T
    h1p, ab = _tc_scale(xw, degt)
    s1p = _sc_segsum(h1p, src2d, dst2d, HIDDEN)
    h2p = _tc_mid(s1p, ab, W2)
    samp = jnp.concatenate([sampled_nodes, jnp.zeros((SP - S,), jnp.int32)])
    g, gab = _sc_segsum_sample(h2p, src2d, dst2d, ab, samp)
    out = _tc_decoder(g[:, :S], gab[:S])
    return out.reshape(2, S * S)
